# baseline (speedup 1.0000x reference)
import functools

import jax
import jax.numpy as jnp
from jax import lax
from jax.experimental import pallas as pl
from jax.experimental.pallas import tpu as pltpu
from jax.experimental.pallas import tpu_sc as plsc

NC, NS, L = 2, 16, 16
NW = NC * NS

B, C, H, W = 8, 5, 64, 2048
IDX_MAX = 64
TW = 128
TPB = NW // B
ROWS_PER_TEC = H // TPB
CR, CC = 8, 512
NRC = ROWS_PER_TEC // CR
NCC = W // CC


def _sc_gather(inp, so):
    mesh = plsc.VectorSubcoreMesh(core_axis_name="c", subcore_axis_name="s",
                                  num_cores=NC, num_subcores=NS)

    @functools.partial(
        pl.kernel,
        mesh=mesh,
        out_type=jax.ShapeDtypeStruct((B, C, H, W), jnp.float32),
        compiler_params=pltpu.CompilerParams(use_tc_tiling_on_sc=True,
                                             needs_layout_passes=False),
        scratch_types=[
            pltpu.VMEM((C, IDX_MAX, TW), jnp.float32),
            pltpu.VMEM((2, CR, CC), jnp.int32),
            pltpu.VMEM((2, CR, CC), jnp.int32),
            pltpu.VMEM((2, C, CR, CC), jnp.float32),
            pltpu.SemaphoreType.DMA,
            pltpu.SemaphoreType.DMA,
            pltpu.SemaphoreType.DMA,
            pltpu.SemaphoreType.DMA,
            pltpu.SemaphoreType.DMA,
        ],
    )
    def k(inp_hbm, so_hbm, out_hbm, table_v, hi_v, wi_v, outc_v,
          tsem, isem0, isem1, osem0, osem1):
        isem = (isem0, isem1)
        osem = (osem0, osem1)
        wid = lax.axis_index("c") * NS + lax.axis_index("s")
        b = wid // TPB
        q = wid % TPB
        row0 = q * ROWS_PER_TEC

        tcopies = [
            pltpu.async_copy(inp_hbm.at[b, c, :, pl.ds(0, TW)],
                             table_v.at[c], tsem)
            for c in range(C)
        ]

        chunks = [(rc, cc) for rc in range(NRC) for cc in range(NCC)]

        def start_idx(chunk_i, buf):
            rc, cc = chunks[chunk_i]
            r = row0 + rc * CR
            col = cc * CC
            return (
                pltpu.async_copy(
                    so_hbm.at[b, 0, pl.ds(r, CR), pl.ds(col, CC)],
                    hi_v.at[buf], isem[buf]),
                pltpu.async_copy(
                    so_hbm.at[b, 1, pl.ds(r, CR), pl.ds(col, CC)],
                    wi_v.at[buf], isem[buf]),
            )

        pend_idx = {0: start_idx(0, 0)}
        for t in tcopies:
            t.wait()

        pend_out = {}
        n_chunks = len(chunks)
        for ch in range(n_chunks):
            buf = ch % 2
            if ch + 1 < n_chunks:
                pend_idx[ch + 1] = start_idx(ch + 1, 1 - buf)
            for cp in pend_idx.pop(ch):
                cp.wait()
            if ch >= 2:
                for cp in pend_out.pop(ch - 2):
                    cp.wait()

            for r in range(CR):
                @plsc.parallel_loop(0, CC // L, 1, unroll=8)
                def body(j, r=r):
                    s = pl.ds(j * L, L)
                    hi = hi_v[buf, r, s]
                    wi = wi_v[buf, r, s]
                    for c in range(C):
                        cs = jnp.full((L,), c, jnp.int32)
                        outc_v[buf, c, r, s] = plsc.load_gather(table_v,
                                                                [cs, hi, wi])

            rc, cc = chunks[ch]
            r = row0 + rc * CR
            col = cc * CC
            pend_out[ch] = tuple(
                pltpu.async_copy(outc_v.at[buf, c],
                                 out_hbm.at[b, c, pl.ds(r, CR),
                                            pl.ds(col, CC)],
                                 osem[buf])
                for c in range(C)
            )
        for cps in pend_out.values():
            for cp in cps:
                cp.wait()

    return k(inp, so)


def kernel(input, sensor_overlap, _scale_h=1, _scale_w=1):
    return _sc_gather(input, sensor_overlap)


# speedup vs baseline: 1.3586x; 1.3586x over previous
import functools

import jax
import jax.numpy as jnp
from jax import lax
from jax.experimental import pallas as pl
from jax.experimental.pallas import tpu as pltpu
from jax.experimental.pallas import tpu_sc as plsc

NC, NS, L = 2, 16, 16
NW = NC * NS

B, C, H, W = 8, 5, 64, 2048
IDX_MAX = 64
TW = 128
TPB = NW // B
ROWS_PER_TEC = H // TPB
CR, CC = 8, 512
NRC = ROWS_PER_TEC // CR
NCC = W // CC


def _sc_gather(inp, so):
    mesh = plsc.VectorSubcoreMesh(core_axis_name="c", subcore_axis_name="s",
                                  num_cores=NC, num_subcores=NS)

    @functools.partial(
        pl.kernel,
        mesh=mesh,
        out_type=jax.ShapeDtypeStruct((B, C, H, W), jnp.float32),
        compiler_params=pltpu.CompilerParams(use_tc_tiling_on_sc=True,
                                             needs_layout_passes=False),
        scratch_types=[
            pltpu.VMEM((C, IDX_MAX, TW), jnp.float32),
            pltpu.VMEM((2, CR, CC), jnp.int32),
            pltpu.VMEM((2, CR, CC), jnp.int32),
            pltpu.VMEM((2, C, CR, CC), jnp.float32),
            pltpu.SemaphoreType.DMA,
            pltpu.SemaphoreType.DMA,
            pltpu.SemaphoreType.DMA,
            pltpu.SemaphoreType.DMA,
            pltpu.SemaphoreType.DMA,
        ],
    )
    def k(inp_hbm, so_hbm, out_hbm, table_v, hi_v, wi_v, outc_v,
          tsem, isem0, isem1, osem0, osem1):
        isem = (isem0, isem1)
        osem = (osem0, osem1)
        wid = lax.axis_index("c") * NS + lax.axis_index("s")
        b = wid // TPB
        q = wid % TPB
        row0 = q * ROWS_PER_TEC

        tcopies = [
            pltpu.async_copy(inp_hbm.at[b, c, :, pl.ds(0, TW)],
                             table_v.at[c], tsem)
            for c in range(C)
        ]

        chunks = [(rc, cc) for rc in range(NRC) for cc in range(NCC)]

        def start_idx(chunk_i, buf):
            rc, cc = chunks[chunk_i]
            r = row0 + rc * CR
            col = cc * CC
            return (
                pltpu.async_copy(
                    so_hbm.at[b, 0, pl.ds(r, CR), pl.ds(col, CC)],
                    hi_v.at[buf], isem[buf]),
                pltpu.async_copy(
                    so_hbm.at[b, 1, pl.ds(r, CR), pl.ds(col, CC)],
                    wi_v.at[buf], isem[buf]),
            )

        pend_idx = {0: start_idx(0, 0)}
        for t in tcopies:
            t.wait()

        pend_out = {}
        n_chunks = len(chunks)
        for ch in range(n_chunks):
            buf = ch % 2
            if ch + 1 < n_chunks:
                pend_idx[ch + 1] = start_idx(ch + 1, 1 - buf)
            for cp in pend_idx.pop(ch):
                cp.wait()
            if ch >= 2:
                for cp in pend_out.pop(ch - 2):
                    cp.wait()

            @plsc.parallel_loop(0, CR * CC // L, 1, unroll=4)
            def body(j):
                r = j // (CC // L)
                s = pl.ds((j % (CC // L)) * L, L)
                hi = hi_v[buf, r, s]
                wi = wi_v[buf, r, s]
                for c in range(C):
                    cs = jnp.full((L,), c, jnp.int32)
                    outc_v[buf, c, r, s] = plsc.load_gather(table_v,
                                                            [cs, hi, wi])

            rc, cc = chunks[ch]
            r = row0 + rc * CR
            col = cc * CC
            pend_out[ch] = tuple(
                pltpu.async_copy(outc_v.at[buf, c],
                                 out_hbm.at[b, c, pl.ds(r, CR),
                                            pl.ds(col, CC)],
                                 osem[buf])
                for c in range(C)
            )
        for cps in pend_out.values():
            for cp in cps:
                cp.wait()

    return k(inp, so)


def kernel(input, sensor_overlap, _scale_h=1, _scale_w=1):
    return _sc_gather(input, sensor_overlap)
